# Initial kernel scaffold; baseline (speedup 1.0000x reference)
import jax
import jax.numpy as jnp
from jax.experimental import pallas as pl


def kernel(x, edge_index, edge_attr, batch, W_l, b_l, W_r, b_r, W_e, att, bias):
    raise NotImplementedError("write your pallas kernel here")


# speedup vs baseline: 10.9076x; 10.9076x over previous
import functools

import jax
import jax.numpy as jnp
from jax import lax
from jax.experimental import pallas as pl
from jax.experimental.pallas import tpu as pltpu
from jax.experimental.pallas import tpu_sc as plsc

_N = 50000
_E = 800000
_G = 64
_NP = 50176
_EP = 819200
_NW = 32
_EW = _EP // _NW
_CH = 1280
_NCHUNK = _EW // _CH
_NGRP = _CH // 16
_NSL = _CH // 128
_NSTR = _NP // 16
_NB = _NP // _NW
_TSL = 112
_CLAMP = 85.0


def _sc_mesh():
    return plsc.VectorSubcoreMesh(
        core_axis_name="c", subcore_axis_name="s", num_cores=2, num_subcores=16
    )


_SC_PARAMS = dict(
    compiler_params=pltpu.CompilerParams(
        needs_layout_passes=False, use_tc_tiling_on_sc=False
    )
)


def _prep_body(ea_ref, x_ref, wl_ref, wr_ref, we_ref, att_ref, bs_ref, out_ref):
    mean_a = jnp.sum(ea_ref[...]) * (1.0 / _E)
    xv = x_ref[...]
    acc0 = jnp.zeros_like(xv)
    acc1 = jnp.zeros_like(xv)
    for c in range(128):
        u = xv * (wl_ref[0, c] + wr_ref[0, c]) + (mean_a * we_ref[0, c] + bs_ref[0, c])
        u = jnp.maximum(u, 0.2 * u)
        t = u * att_ref[0, c]
        if c < 64:
            acc0 = acc0 + t
        else:
            acc1 = acc1 + t
    b0 = lax.bitcast_convert_type(acc0, jnp.uint32)
    b1 = lax.bitcast_convert_type(acc1, jnp.uint32)
    pk = (b1 & jnp.uint32(0xFFFF0000)) | (b0 >> 16)
    out_ref[...] = lax.bitcast_convert_type(pk, jnp.int32)


def _prep(ea2, x2, wl2, wr2, we2, att2, bs2):
    smem = pl.BlockSpec(memory_space=pltpu.SMEM)
    return pl.pallas_call(
        _prep_body,
        out_shape=jax.ShapeDtypeStruct((392, 128), jnp.int32),
        in_specs=[pl.BlockSpec((6250, 128), lambda: (0, 0)),
                  pl.BlockSpec((392, 128), lambda: (0, 0)),
                  smem, smem, smem, smem, smem],
        out_specs=pl.BlockSpec((392, 128), lambda: (0, 0)),
    )(ea2, x2, wl2, wr2, we2, att2, bs2)


def _gather_sc(xp, srcp, dstp):
    @functools.partial(
        pl.kernel,
        out_type=[jax.ShapeDtypeStruct((_EP,), jnp.float32),
                  jax.ShapeDtypeStruct((_EP,), jnp.float32)],
        mesh=_sc_mesh(),
        scratch_types=[pltpu.VMEM((_NP,), jnp.float32),
                       pltpu.VMEM((_CH,), jnp.int32),
                       pltpu.VMEM((_CH,), jnp.float32),
                       pltpu.VMEM((_CH,), jnp.float32)],
        **_SC_PARAMS,
    )
    def k(x_hbm, src_hbm, dst_hbm, s_out, d_out, xt, idxv, sv, dv):
        wid = lax.axis_index("s") * 2 + lax.axis_index("c")
        pltpu.sync_copy(x_hbm, xt)

        def chunk(ci, carry):
            off = wid * _EW + ci * _CH
            pltpu.sync_copy(src_hbm.at[pl.ds(off, _CH)], idxv)
            for g in range(_NGRP):
                sl = pl.ds(g * 16, 16)
                sv[sl] = plsc.load_gather(xt, [idxv[sl]])
            pltpu.sync_copy(sv, s_out.at[pl.ds(off, _CH)])
            pltpu.sync_copy(dst_hbm.at[pl.ds(off, _CH)], idxv)
            for g in range(_NGRP):
                sl = pl.ds(g * 16, 16)
                dv[sl] = plsc.load_gather(xt, [idxv[sl]])
            pltpu.sync_copy(dv, d_out.at[pl.ds(off, _CH)])
            return carry

        lax.fori_loop(0, _NCHUNK, chunk, 0)

    return k(xp, srcp, dstp)


def _alpha_body(wl_ref, wr_ref, we_ref, att_ref, bs_ref, s_ref, d_ref, a_ref,
                o0_ref, o1_ref):
    sv = s_ref[...]
    dv = d_ref[...]
    av = a_ref[...]
    acc0 = jnp.zeros_like(sv)
    acc1 = jnp.zeros_like(sv)
    for c in range(128):
        u = sv * wl_ref[0, c] + dv * wr_ref[0, c] + av * we_ref[0, c] + bs_ref[0, c]
        u = jnp.maximum(u, 0.2 * u)
        t = u * att_ref[0, c]
        if c < 64:
            acc0 = acc0 + t
        else:
            acc1 = acc1 + t
    o0_ref[...] = acc0
    o1_ref[...] = acc1


def _alpha(s2, d2, a2, wl2, wr2, we2, att2, bs2):
    smem = pl.BlockSpec(memory_space=pltpu.SMEM)
    blk = pl.BlockSpec((64, 128), lambda i: (i, 0))
    return pl.pallas_call(
        _alpha_body,
        grid=(100,),
        out_shape=[jax.ShapeDtypeStruct((6400, 128), jnp.float32),
                   jax.ShapeDtypeStruct((6400, 128), jnp.float32)],
        in_specs=[smem, smem, smem, smem, smem, blk, blk, blk],
        out_specs=[blk, blk],
    )(wl2, wr2, we2, att2, bs2, s2, d2, a2)


def _scatter_sc(refpk_in, dstp, al0, al1, s_e, zeros4):
    @functools.partial(
        pl.kernel,
        out_type=jax.ShapeDtypeStruct((2 * _NP, 4), jnp.float32),
        mesh=_sc_mesh(),
        scratch_types=[pltpu.VMEM((_NP,), jnp.int32),
                       pltpu.VMEM((_CH,), jnp.int32),
                       pltpu.VMEM((128,), jnp.int32),
                       pltpu.VMEM((_CH,), jnp.float32),
                       pltpu.VMEM((_CH,), jnp.float32),
                       pltpu.VMEM((_CH,), jnp.float32),
                       pltpu.VMEM((_CH, 4), jnp.float32),
                       pltpu.VMEM((_NSTR, 4), jnp.float32),
                       pltpu.VMEM_SHARED((_NP, 4), jnp.float32)],
        **_SC_PARAMS,
    )
    def k(refpk_hbm, dst_hbm, a0_hbm, a1_hbm, s_hbm, zero_hbm,
          parts_out, refpk, dstv, dst2v, a0v, a1v, sv, rows, stage, acc):
        cid = lax.axis_index("c")
        sid = lax.axis_index("s")
        wid = sid * 2 + cid
        pltpu.sync_copy(refpk_hbm, refpk)
        pltpu.sync_copy(zero_hbm.at[pl.ds(sid * _NSTR, _NSTR)], stage)
        pltpu.sync_copy(stage, acc.at[pl.ds(sid * _NSTR, _NSTR)])
        plsc.subcore_barrier()
        lanes = lax.iota(jnp.int32, 16)
        cols = [jnp.full((16,), kk, jnp.int32) for kk in range(4)]

        def chunk(ci, carry):
            off = wid * _EW + ci * _CH
            pltpu.sync_copy(dst_hbm.at[pl.ds(off, _CH)], dstv)
            pltpu.sync_copy(a0_hbm.at[pl.ds(off, _CH)], a0v)
            pltpu.sync_copy(a1_hbm.at[pl.ds(off, _CH)], a1v)
            pltpu.sync_copy(s_hbm.at[pl.ds(off, _CH)], sv)
            for g in range(_NGRP):
                sl = pl.ds(g * 16, 16)
                w = plsc.load_gather(refpk, [dstv[sl]])
                r0 = plsc.bitcast(w << 16, jnp.float32)
                r1 = plsc.bitcast(w & jnp.int32(-65536), jnp.float32)
                p0 = jnp.exp(jnp.minimum(a0v[sl] - r0, _CLAMP))
                p1 = jnp.exp(jnp.minimum(a1v[sl] - r1, _CLAMP))
                se = sv[sl]
                ridx = lanes + (g * 16)
                plsc.store_scatter(rows, [ridx, cols[0]], p0)
                plsc.store_scatter(rows, [ridx, cols[1]], p1)
                plsc.store_scatter(rows, [ridx, cols[2]], p0 * se)
                plsc.store_scatter(rows, [ridx, cols[3]], p1 * se)
            for j in range(_NSL):
                for kk in range(8):
                    dst2v[pl.ds(kk * 16, 16)] = dstv[pl.ds(j * 128 + kk * 16, 16)]
                pltpu.sync_copy(rows.at[pl.ds(j * 128, 128)],
                                acc.at[dst2v], add=True)
            return carry

        lax.fori_loop(0, _NCHUNK, chunk, 0)
        plsc.subcore_barrier()
        pltpu.sync_copy(acc.at[pl.ds(sid * _NSTR, _NSTR)], stage)
        pltpu.sync_copy(stage, parts_out.at[pl.ds(cid * _NP + sid * _NSTR, _NSTR)])

    return k(refpk_in, dstp, al0, al1, s_e, zeros4)


def _final_body(tp_ref, wl_ref, bl_ref, bias_ref, o_ref):
    t = tp_ref[0:72, :] + tp_ref[72:144, :]
    t = t[0:64, :]
    wl = wl_ref[...]
    bl = bl_ref[...]
    bias = bias_ref[...]
    o_ref[...] = 0.5 * (t[:, 0:1] * wl[:, 0:64] + t[:, 1:2] * wl[:, 64:128]
                        + t[:, 2:3] * bl[:, 0:64] + t[:, 3:4] * bl[:, 64:128]) \
        + t[:, 4:5] * bias


def _final(tparts, wl2, bl2, bias2):
    return pl.pallas_call(
        _final_body,
        out_shape=jax.ShapeDtypeStruct((_G, 64), jnp.float32),
    )(tparts, wl2, bl2, bias2)


def kernel(x, edge_index, edge_attr, batch, W_l, b_l, W_r, b_r, W_e, att, bias):
    f32 = jnp.float32
    i32 = jnp.int32
    x1 = x[:, 0].astype(f32)
    xp = jnp.concatenate([x1, jnp.zeros((_NP - _N,), f32)])
    srcp = jnp.concatenate([edge_index[0].astype(i32),
                            jnp.zeros((_EP - _E,), i32)])
    dstp = jnp.concatenate([edge_index[1].astype(i32),
                            jnp.full((_EP - _E,), _N, i32)])
    eap = jnp.concatenate([edge_attr[:, 0].astype(f32),
                           jnp.zeros((_EP - _E,), f32)])
    batchp = jnp.concatenate([batch.astype(i32), jnp.full((_NP - _N,), _G, i32)])
    wl2 = W_l.reshape(1, 128).astype(f32)
    wr2 = W_r.reshape(1, 128).astype(f32)
    we2 = W_e.reshape(1, 128).astype(f32)
    att2 = att.reshape(1, 128).astype(f32)
    bs2 = (b_l + b_r).reshape(1, 128).astype(f32)
    bl2 = b_l.reshape(1, 128).astype(f32)
    bias2 = bias.reshape(1, 64).astype(f32)

    refpack = _prep(edge_attr.reshape(6250, 128).astype(f32),
                    xp.reshape(392, 128), wl2, wr2, we2, att2, bs2)
    s_e, d_e = _gather_sc(xp, srcp, dstp)
    al0, al1 = _alpha(s_e.reshape(6400, 128), d_e.reshape(6400, 128),
                      eap.reshape(6400, 128), wl2, wr2, we2, att2, bs2)
    rp = refpack.reshape(_NP)
    al0 = al0.reshape(_EP)
    al1 = al1.reshape(_EP)
    w = rp[dstp]
    r0 = lax.bitcast_convert_type(w << 16, f32)
    r1 = lax.bitcast_convert_type(w & jnp.int32(-65536), f32)
    p0 = jnp.exp(jnp.minimum(al0 - r0, _CLAMP))
    p1 = jnp.exp(jnp.minimum(al1 - r1, _CLAMP))
    st = jnp.zeros((_NP, 4), f32).at[dstp].add(
        jnp.stack([p0, p1, p0 * s_e, p1 * s_e], axis=1))
    den0 = st[:, 0] + 1.0
    den1 = st[:, 1] + 1.0
    num0 = st[:, 2] + xp
    num1 = st[:, 3] + xp
    zcol = jnp.zeros((_NP,), f32)
    rowst = jnp.stack([num0 / (den0 + 1e-16), num1 / (den1 + 1e-16),
                       den0 / (den0 + 1e-16), den1 / (den1 + 1e-16),
                       jnp.ones((_NP,), f32), zcol, zcol, zcol], axis=1)
    tacc = jnp.zeros((72, 8), f32).at[batchp].add(rowst)
    tparts = jnp.concatenate([tacc, jnp.zeros((72, 8), f32)], axis=0)
    return _final(tparts, wl2, bl2, bias2)
